# baseline (speedup 1.0000x reference)
import functools

import jax
import jax.numpy as jnp
from jax import lax
from jax.experimental import pallas as pl
from jax.experimental.pallas import tpu as pltpu
from jax.experimental.pallas import tpu_sc as plsc

NC = 2
NS = 16
NW = NC * NS
LANES = 16
CHUNK = 128
WT = 5
WCOLS = WT * 128
TAILPAD = 128


def _make_extract_kernel(B, F, n_cols_e, n_cols_w):
    per_w = B // NW
    n_blk = per_w // LANES
    mesh = plsc.VectorSubcoreMesh(core_axis_name="c", subcore_axis_name="s")

    def geom(n_cols):
        n_tiles = n_cols // CHUNK
        return n_tiles * CHUNK, max(n_tiles - WT, 0)

    main_e, tmax_e = geom(n_cols_e)
    main_w, tmax_w = geom(n_cols_w)

    @functools.partial(
        pl.kernel,
        out_type=(jax.ShapeDtypeStruct((B * F,), jnp.float32),
                  jax.ShapeDtypeStruct((B * F,), jnp.float32)),
        mesh=mesh,
        scratch_types=[
            pltpu.VMEM((2, F, WCOLS), jnp.float32),
            pltpu.VMEM((per_w * F,), jnp.float32),
            pltpu.VMEM((per_w,), jnp.int32),
            pltpu.VMEM((F, TAILPAD), jnp.float32),
            pltpu.SemaphoreType.DMA,
        ],
        compiler_params=pltpu.CompilerParams(needs_layout_passes=False,
                                             use_tc_tiling_on_sc=True),
    )
    def extract(et_hbm, w_hbm, etail_hbm, wtail_hbm, rs_hbm, cs_hbm,
                eout_hbm, wout_hbm, wbuf, stg, sc_v, tbuf, sem):
        cid = lax.axis_index("c")
        sid = lax.axis_index("s")
        wid = sid * NC + cid
        iota = lax.broadcasted_iota(jnp.int32, (LANES,), 0)

        for tab, tail, srt, out, main_c, t_max in (
                (et_hbm, etail_hbm, rs_hbm, eout_hbm, main_e, tmax_e),
                (w_hbm, wtail_hbm, cs_hbm, wout_hbm, main_w, tmax_w)):
            pltpu.sync_copy(srt.at[pl.ds(wid * per_w, per_w)], sc_v)
            pltpu.sync_copy(tail, tbuf)

            def drain():
                pltpu.make_async_copy(tab.at[:, pl.ds(0, WCOLS)],
                                      wbuf.at[0], sem).wait()

            def blk_body(blk, wstate):
                cols16 = sc_v[pl.ds(blk * LANES, LANES)]
                sidx0 = (blk * LANES + iota) * F
                tmask = cols16 >= main_c
                n_tail = jnp.sum(tmask.astype(jnp.int32))

                @pl.when(n_tail > 0)
                def _():
                    tloc = jnp.clip(cols16 - main_c, 0, TAILPAD - 1)
                    fv = jnp.zeros((LANES,), jnp.int32)
                    sidx = sidx0
                    for _f in range(F):
                        tv = plsc.load_gather(tbuf, [fv, tloc], mask=tmask)
                        plsc.store_scatter(stg, [sidx], tv, mask=tmask)
                        fv = fv + 1
                        sidx = sidx + 1

                def wcond(state):
                    done = state[0]
                    return jnp.sum(done.astype(jnp.int32)) < LANES

                def wbody(state):
                    done, t0_old, cur_old, t0_pref = state
                    cbig = jnp.where(done, jnp.int32(1 << 30), cols16)
                    cmin = jnp.min(cbig)
                    hit = jnp.logical_and(cmin >= t0_old * CHUNK,
                                          cmin < t0_old * CHUNK + WCOLS)
                    t0_need = jnp.where(hit, t0_old,
                                        jnp.minimum(cmin >> 7, t_max))
                    miss = jnp.logical_not(hit)
                    pref_ok = jnp.logical_and(miss, t0_pref == t0_need)
                    cur = jnp.where(pref_ok, 1 - cur_old, cur_old)

                    @pl.when(jnp.logical_and(miss, t0_pref >= 0))
                    def _():
                        drain()

                    @pl.when(jnp.logical_and(miss, t0_pref != t0_need))
                    def _():
                        pltpu.sync_copy(
                            tab.at[:, pl.ds(t0_need * CHUNK, WCOLS)],
                            wbuf.at[cur])

                    t0_next = jnp.minimum(t0_need + WT, t_max)

                    @pl.when(miss)
                    def _():
                        pltpu.async_copy(
                            tab.at[:, pl.ds(t0_next * CHUNK, WCOLS)],
                            wbuf.at[1 - cur], sem)

                    c0 = t0_need * CHUNK
                    cover = jnp.logical_and(
                        jnp.logical_and(cols16 >= c0, cols16 < c0 + WCOLS),
                        jnp.logical_not(tmask))
                    lc = jnp.clip(cols16 - c0, 0, WCOLS - 1)
                    cvec = jnp.zeros((LANES,), jnp.int32) + cur
                    fv = jnp.zeros((LANES,), jnp.int32)
                    sidx = sidx0
                    for _f in range(F):
                        v = plsc.load_gather(wbuf, [cvec, fv, lc], mask=cover)
                        plsc.store_scatter(stg, [sidx], v, mask=cover)
                        fv = fv + 1
                        sidx = sidx + 1
                    return (jnp.logical_or(done, cover), t0_need, cur,
                            jnp.where(miss, t0_next, t0_pref))

                st = lax.while_loop(wcond, wbody, (tmask,) + wstate)
                return st[1:]

            wstate = lax.fori_loop(
                0, n_blk, blk_body,
                (jnp.int32(-(1 << 20)), jnp.int32(0), jnp.int32(-1)))

            @pl.when(wstate[2] >= 0)
            def _():
                drain()

            pltpu.sync_copy(stg, out.at[pl.ds(wid * per_w * F, per_w * F)])

    return extract


def _make_join_kernel(B, F):
    n_chunks = B // NW // CHUNK
    mesh = plsc.VectorSubcoreMesh(core_axis_name="c", subcore_axis_name="s")

    @functools.partial(
        pl.kernel,
        out_type=jax.ShapeDtypeStruct((B,), jnp.float32),
        mesh=mesh,
        scratch_types=[
            pltpu.VMEM((CHUNK,), jnp.int32),
            pltpu.VMEM((CHUNK,), jnp.int32),
            pltpu.VMEM((CHUNK, F), jnp.float32),
            pltpu.VMEM((CHUNK, F), jnp.float32),
            pltpu.VMEM((CHUNK,), jnp.float32),
            pltpu.SemaphoreType.DMA,
        ],
        compiler_params=pltpu.CompilerParams(needs_layout_passes=False,
                                             use_tc_tiling_on_sc=False),
    )
    def join(eidx_hbm, widx_hbm, es_hbm, ws_hbm, out_hbm,
             eidx_v, widx_v, e_v, w_v, out_v, sem):
        cid = lax.axis_index("c")
        sid = lax.axis_index("s")
        wid = sid * NC + cid
        iota = lax.broadcasted_iota(jnp.int32, (LANES,), 0)

        for g in range(n_chunks):
            base = (wid * n_chunks + g) * CHUNK
            pltpu.sync_copy(eidx_hbm.at[pl.ds(base, CHUNK)], eidx_v)
            pltpu.sync_copy(widx_hbm.at[pl.ds(base, CHUNK)], widx_v)
            cp_e = pltpu.async_copy(es_hbm.at[eidx_v], e_v, sem)
            cp_w = pltpu.async_copy(ws_hbm.at[widx_v], w_v, sem)
            cp_e.wait()
            cp_w.wait()

            def block_body(bi, _):
                b0 = bi * LANES
                rows16 = b0 + iota
                acc = jnp.zeros((LANES,), jnp.float32)
                cols16 = jnp.zeros((LANES,), jnp.int32)
                for _f in range(F):
                    e_vals = plsc.load_gather(e_v, [rows16, cols16])
                    w_vals = plsc.load_gather(w_v, [rows16, cols16])
                    acc = acc + e_vals * w_vals
                    cols16 = cols16 + 1
                out_v[pl.ds(b0, LANES)] = acc
                return 0

            lax.fori_loop(0, CHUNK // LANES, block_body, 0)
            pltpu.sync_copy(out_v, out_hbm.at[pl.ds(base, CHUNK)])

    return join


def _padded_tail(tab_fmajor, main_c):
    tail = tab_fmajor[:, main_c:]
    return jnp.pad(tail, ((0, 0), (0, TAILPAD - tail.shape[1])))


def kernel(batch, E, W):
    B = batch.shape[0]
    n_ent, F = E.shape
    n_words = W.shape[1]
    rows = batch[:, 0].astype(jnp.int32)
    cols = batch[:, 1].astype(jnp.int32)

    order_e = jnp.argsort(rows)
    order_w = jnp.argsort(cols)
    rs = jnp.take(rows, order_e)
    cs = jnp.take(cols, order_w)
    ar = jnp.arange(B, dtype=jnp.int32)
    inv_e = jnp.zeros((B,), jnp.int32).at[order_e].set(ar)
    inv_w = jnp.zeros((B,), jnp.int32).at[order_w].set(ar)

    ET = E.T
    main_e = (n_ent // CHUNK) * CHUNK
    main_w = (n_words // CHUNK) * CHUNK
    etail = _padded_tail(ET, main_e)
    wtail = _padded_tail(W, main_w)

    eflat, wflat = _make_extract_kernel(B, F, n_ent, n_words)(
        ET, W, etail, wtail, rs, cs)
    esamp = eflat.reshape(B, F)
    wsamp = wflat.reshape(B, F)
    return _make_join_kernel(B, F)(inv_e, inv_w, esamp, wsamp)


# speedup vs baseline: 1.0605x; 1.0605x over previous
import functools

import jax
import jax.numpy as jnp
from jax import lax
from jax.experimental import pallas as pl
from jax.experimental.pallas import tpu as pltpu
from jax.experimental.pallas import tpu_sc as plsc

NC = 2
NS = 16
NW = NC * NS
LANES = 16
CHUNK = 128
WT = 10
WCOLS = WT * 128
TAILPAD = 128


def _make_extract_kernel(B, F, n_cols_e, n_cols_w):
    per_w = B // NW
    n_blk = per_w // LANES
    mesh = plsc.VectorSubcoreMesh(core_axis_name="c", subcore_axis_name="s")

    def geom(n_cols):
        n_tiles = n_cols // CHUNK
        return n_tiles * CHUNK, max(n_tiles - WT, 0)

    main_e, tmax_e = geom(n_cols_e)
    main_w, tmax_w = geom(n_cols_w)

    @functools.partial(
        pl.kernel,
        out_type=(jax.ShapeDtypeStruct((B * F,), jnp.float32),
                  jax.ShapeDtypeStruct((B * F,), jnp.float32)),
        mesh=mesh,
        scratch_types=[
            pltpu.VMEM((F, WCOLS), jnp.float32),
            pltpu.VMEM((per_w * F,), jnp.float32),
            pltpu.VMEM((per_w,), jnp.int32),
            pltpu.VMEM((F, TAILPAD), jnp.float32),
        ],
        compiler_params=pltpu.CompilerParams(needs_layout_passes=False,
                                             use_tc_tiling_on_sc=True),
    )
    def extract(et_hbm, w_hbm, etail_hbm, wtail_hbm, rs_hbm, cs_hbm,
                eout_hbm, wout_hbm, wbuf, stg, sc_v, tbuf):
        cid = lax.axis_index("c")
        sid = lax.axis_index("s")
        wid = sid * NC + cid
        iota = lax.broadcasted_iota(jnp.int32, (LANES,), 0)

        for tab, tail, srt, out, main_c, t_max in (
                (et_hbm, etail_hbm, rs_hbm, eout_hbm, main_e, tmax_e),
                (w_hbm, wtail_hbm, cs_hbm, wout_hbm, main_w, tmax_w)):
            pltpu.sync_copy(srt.at[pl.ds(wid * per_w, per_w)], sc_v)
            pltpu.sync_copy(tail, tbuf)

            def blk_body(blk, t0_cur):
                cols16 = sc_v[pl.ds(blk * LANES, LANES)]
                sidx0 = (blk * LANES + iota) * F
                tmask = cols16 >= main_c
                n_tail = jnp.sum(tmask.astype(jnp.int32))

                @pl.when(n_tail > 0)
                def _():
                    tloc = jnp.clip(cols16 - main_c, 0, TAILPAD - 1)
                    fv = jnp.zeros((LANES,), jnp.int32)
                    sidx = sidx0
                    for _f in range(F):
                        tv = plsc.load_gather(tbuf, [fv, tloc], mask=tmask)
                        plsc.store_scatter(stg, [sidx], tv, mask=tmask)
                        fv = fv + 1
                        sidx = sidx + 1

                def wcond(state):
                    done, _ = state
                    return jnp.sum(done.astype(jnp.int32)) < LANES

                def wbody(state):
                    done, t0_old = state
                    cbig = jnp.where(done, jnp.int32(1 << 30), cols16)
                    cmin = jnp.min(cbig)
                    hit = jnp.logical_and(cmin >= t0_old * CHUNK,
                                          cmin < t0_old * CHUNK + WCOLS)
                    t0 = jnp.where(hit, t0_old,
                                   jnp.minimum(cmin >> 7, t_max))
                    c0 = t0 * CHUNK

                    @pl.when(jnp.logical_not(hit))
                    def _():
                        pltpu.sync_copy(tab.at[:, pl.ds(c0, WCOLS)], wbuf)

                    cover = jnp.logical_and(
                        jnp.logical_and(cols16 >= c0, cols16 < c0 + WCOLS),
                        jnp.logical_not(tmask))
                    lc = jnp.clip(cols16 - c0, 0, WCOLS - 1)
                    fv = jnp.zeros((LANES,), jnp.int32)
                    sidx = sidx0
                    for _f in range(F):
                        v = plsc.load_gather(wbuf, [fv, lc], mask=cover)
                        plsc.store_scatter(stg, [sidx], v, mask=cover)
                        fv = fv + 1
                        sidx = sidx + 1
                    return jnp.logical_or(done, cover), t0

                _, t0_out = lax.while_loop(wcond, wbody, (tmask, t0_cur))
                return t0_out

            lax.fori_loop(0, n_blk, blk_body, jnp.int32(-(1 << 20)))
            pltpu.sync_copy(stg, out.at[pl.ds(wid * per_w * F, per_w * F)])

    return extract


def _make_join_kernel(B, F):
    per_w = B // NW
    n_chunks = per_w // CHUNK
    mesh = plsc.VectorSubcoreMesh(core_axis_name="c", subcore_axis_name="s")

    @functools.partial(
        pl.kernel,
        out_type=jax.ShapeDtypeStruct((B,), jnp.float32),
        mesh=mesh,
        scratch_types=[
            pltpu.VMEM((per_w,), jnp.int32),
            pltpu.VMEM((per_w,), jnp.int32),
            pltpu.VMEM((per_w, F), jnp.float32),
            pltpu.VMEM((per_w, F), jnp.float32),
            pltpu.VMEM((per_w,), jnp.float32),
            pltpu.SemaphoreType.DMA,
        ],
        compiler_params=pltpu.CompilerParams(needs_layout_passes=False,
                                             use_tc_tiling_on_sc=False),
    )
    def join(eidx_hbm, widx_hbm, es_hbm, ws_hbm, out_hbm,
             eidx_v, widx_v, e_v, w_v, out_v, sem):
        cid = lax.axis_index("c")
        sid = lax.axis_index("s")
        wid = sid * NC + cid
        iota = lax.broadcasted_iota(jnp.int32, (LANES,), 0)
        base = wid * per_w

        pltpu.sync_copy(eidx_hbm.at[pl.ds(base, per_w)], eidx_v)
        pltpu.sync_copy(widx_hbm.at[pl.ds(base, per_w)], widx_v)
        cps = []
        for g in range(n_chunks):
            o = g * CHUNK
            cps.append(pltpu.async_copy(
                es_hbm.at[eidx_v.at[pl.ds(o, CHUNK)]],
                e_v.at[pl.ds(o, CHUNK), :], sem))
            cps.append(pltpu.async_copy(
                ws_hbm.at[widx_v.at[pl.ds(o, CHUNK)]],
                w_v.at[pl.ds(o, CHUNK), :], sem))
        for cp in cps:
            cp.wait()

        def block_body(bi, _):
            b0 = bi * LANES
            rows16 = b0 + iota
            acc = jnp.zeros((LANES,), jnp.float32)
            cols16 = jnp.zeros((LANES,), jnp.int32)
            for _f in range(F):
                e_vals = plsc.load_gather(e_v, [rows16, cols16])
                w_vals = plsc.load_gather(w_v, [rows16, cols16])
                acc = acc + e_vals * w_vals
                cols16 = cols16 + 1
            out_v[pl.ds(b0, LANES)] = acc
            return 0

        lax.fori_loop(0, per_w // LANES, block_body, 0)
        pltpu.sync_copy(out_v, out_hbm.at[pl.ds(base, per_w)])

    return join


def _padded_tail(tab_fmajor, main_c):
    tail = tab_fmajor[:, main_c:]
    return jnp.pad(tail, ((0, 0), (0, TAILPAD - tail.shape[1])))


def kernel(batch, E, W):
    B = batch.shape[0]
    n_ent, F = E.shape
    n_words = W.shape[1]
    rows = batch[:, 0].astype(jnp.int32)
    cols = batch[:, 1].astype(jnp.int32)

    order_e = jnp.argsort(rows)
    order_w = jnp.argsort(cols)
    rs = jnp.take(rows, order_e)
    cs = jnp.take(cols, order_w)
    ar = jnp.arange(B, dtype=jnp.int32)
    inv_e = jnp.zeros((B,), jnp.int32).at[order_e].set(ar)
    inv_w = jnp.zeros((B,), jnp.int32).at[order_w].set(ar)

    ET = E.T
    main_e = (n_ent // CHUNK) * CHUNK
    main_w = (n_words // CHUNK) * CHUNK
    etail = _padded_tail(ET, main_e)
    wtail = _padded_tail(W, main_w)

    eflat, wflat = _make_extract_kernel(B, F, n_ent, n_words)(
        ET, W, etail, wtail, rs, cs)
    esamp = eflat.reshape(B, F)
    wsamp = wflat.reshape(B, F)
    return _make_join_kernel(B, F)(inv_e, inv_w, esamp, wsamp)


# speedup vs baseline: 1.0763x; 1.0150x over previous
import functools

import jax
import jax.numpy as jnp
from jax import lax
from jax.experimental import pallas as pl
from jax.experimental.pallas import tpu as pltpu
from jax.experimental.pallas import tpu_sc as plsc

NC = 2
NS = 16
NW = NC * NS
LANES = 16
CHUNK = 128
WT = 10
WCOLS = WT * 128
TAILPAD = 128


def _make_extract_kernel(B, F, n_cols):
    per_w = B // NW
    n_blk = per_w // LANES
    mesh = plsc.VectorSubcoreMesh(core_axis_name="c", subcore_axis_name="s")

    n_tiles = n_cols // CHUNK
    main_c = n_tiles * CHUNK
    t_max = max(n_tiles - WT, 0)

    @functools.partial(
        pl.kernel,
        out_type=jax.ShapeDtypeStruct((B * F,), jnp.float32),
        mesh=mesh,
        scratch_types=[
            pltpu.VMEM((F, WCOLS), jnp.float32),
            pltpu.VMEM((per_w * F,), jnp.float32),
            pltpu.VMEM((per_w,), jnp.int32),
            pltpu.VMEM((F, TAILPAD), jnp.float32),
        ],
        compiler_params=pltpu.CompilerParams(needs_layout_passes=False,
                                             use_tc_tiling_on_sc=True),
    )
    def extract(tab, tail, srt, out, wbuf, stg, sc_v, tbuf):
        cid = lax.axis_index("c")
        sid = lax.axis_index("s")
        wid = sid * NC + cid
        iota = lax.broadcasted_iota(jnp.int32, (LANES,), 0)

        if True:
            pltpu.sync_copy(srt.at[pl.ds(wid * per_w, per_w)], sc_v)
            pltpu.sync_copy(tail, tbuf)

            def blk_body(blk, t0_cur):
                cols16 = sc_v[pl.ds(blk * LANES, LANES)]
                sidx0 = (blk * LANES + iota) * F
                tmask = cols16 >= main_c
                n_tail = jnp.sum(tmask.astype(jnp.int32))

                @pl.when(n_tail > 0)
                def _():
                    tloc = jnp.clip(cols16 - main_c, 0, TAILPAD - 1)
                    fv = jnp.zeros((LANES,), jnp.int32)
                    sidx = sidx0
                    for _f in range(F):
                        tv = plsc.load_gather(tbuf, [fv, tloc], mask=tmask)
                        plsc.store_scatter(stg, [sidx], tv, mask=tmask)
                        fv = fv + 1
                        sidx = sidx + 1

                def wcond(state):
                    done, _ = state
                    return jnp.sum(done.astype(jnp.int32)) < LANES

                def wbody(state):
                    done, t0_old = state
                    cbig = jnp.where(done, jnp.int32(1 << 30), cols16)
                    cmin = jnp.min(cbig)
                    hit = jnp.logical_and(cmin >= t0_old * CHUNK,
                                          cmin < t0_old * CHUNK + WCOLS)
                    t0 = jnp.where(hit, t0_old,
                                   jnp.minimum(cmin >> 7, t_max))
                    c0 = t0 * CHUNK

                    @pl.when(jnp.logical_not(hit))
                    def _():
                        pltpu.sync_copy(tab.at[:, pl.ds(c0, WCOLS)], wbuf)

                    cover = jnp.logical_and(
                        jnp.logical_and(cols16 >= c0, cols16 < c0 + WCOLS),
                        jnp.logical_not(tmask))
                    lc = jnp.clip(cols16 - c0, 0, WCOLS - 1)
                    fv = jnp.zeros((LANES,), jnp.int32)
                    sidx = sidx0
                    for _f in range(F):
                        v = plsc.load_gather(wbuf, [fv, lc], mask=cover)
                        plsc.store_scatter(stg, [sidx], v, mask=cover)
                        fv = fv + 1
                        sidx = sidx + 1
                    return jnp.logical_or(done, cover), t0

                _, t0_out = lax.while_loop(wcond, wbody, (tmask, t0_cur))
                return t0_out

            lax.fori_loop(0, n_blk, blk_body, jnp.int32(-(1 << 20)))
            pltpu.sync_copy(stg, out.at[pl.ds(wid * per_w * F, per_w * F)])

    return extract


def _make_join_kernel(B, F):
    per_w = B // NW
    n_chunks = per_w // CHUNK
    mesh = plsc.VectorSubcoreMesh(core_axis_name="c", subcore_axis_name="s")

    @functools.partial(
        pl.kernel,
        out_type=jax.ShapeDtypeStruct((B,), jnp.float32),
        mesh=mesh,
        scratch_types=[
            pltpu.VMEM((per_w,), jnp.int32),
            pltpu.VMEM((per_w,), jnp.int32),
            pltpu.VMEM((per_w, F), jnp.float32),
            pltpu.VMEM((per_w, F), jnp.float32),
            pltpu.VMEM((per_w,), jnp.float32),
            pltpu.SemaphoreType.DMA,
        ],
        compiler_params=pltpu.CompilerParams(needs_layout_passes=False,
                                             use_tc_tiling_on_sc=False),
    )
    def join(eidx_hbm, widx_hbm, es_hbm, ws_hbm, out_hbm,
             eidx_v, widx_v, e_v, w_v, out_v, sem):
        cid = lax.axis_index("c")
        sid = lax.axis_index("s")
        wid = sid * NC + cid
        iota = lax.broadcasted_iota(jnp.int32, (LANES,), 0)
        base = wid * per_w

        pltpu.sync_copy(eidx_hbm.at[pl.ds(base, per_w)], eidx_v)
        pltpu.sync_copy(widx_hbm.at[pl.ds(base, per_w)], widx_v)
        cps = []
        for g in range(n_chunks):
            o = g * CHUNK
            cps.append(pltpu.async_copy(
                es_hbm.at[eidx_v.at[pl.ds(o, CHUNK)]],
                e_v.at[pl.ds(o, CHUNK), :], sem))
            cps.append(pltpu.async_copy(
                ws_hbm.at[widx_v.at[pl.ds(o, CHUNK)]],
                w_v.at[pl.ds(o, CHUNK), :], sem))
        for cp in cps:
            cp.wait()

        def block_body(bi, _):
            b0 = bi * LANES
            rows16 = b0 + iota
            acc = jnp.zeros((LANES,), jnp.float32)
            cols16 = jnp.zeros((LANES,), jnp.int32)
            for _f in range(F):
                e_vals = plsc.load_gather(e_v, [rows16, cols16])
                w_vals = plsc.load_gather(w_v, [rows16, cols16])
                acc = acc + e_vals * w_vals
                cols16 = cols16 + 1
            out_v[pl.ds(b0, LANES)] = acc
            return 0

        lax.fori_loop(0, per_w // LANES, block_body, 0)
        pltpu.sync_copy(out_v, out_hbm.at[pl.ds(base, per_w)])

    return join


def _padded_tail(tab_fmajor, main_c):
    tail = tab_fmajor[:, main_c:]
    return jnp.pad(tail, ((0, 0), (0, TAILPAD - tail.shape[1])))


def kernel(batch, E, W):
    B = batch.shape[0]
    n_ent, F = E.shape
    n_words = W.shape[1]
    rows = batch[:, 0].astype(jnp.int32)
    cols = batch[:, 1].astype(jnp.int32)

    ar = jnp.arange(B, dtype=jnp.int32)
    rs, order_e = lax.sort((rows, ar), num_keys=1)
    cs, order_w = lax.sort((cols, ar), num_keys=1)
    inv_e = jnp.zeros((B,), jnp.int32).at[order_e].set(ar)
    inv_w = jnp.zeros((B,), jnp.int32).at[order_w].set(ar)

    ET = E.T
    main_e = (n_ent // CHUNK) * CHUNK
    main_w = (n_words // CHUNK) * CHUNK
    etail = _padded_tail(ET, main_e)
    wtail = _padded_tail(W, main_w)

    eflat = _make_extract_kernel(B, F, n_ent)(ET, etail, rs)
    wflat = _make_extract_kernel(B, F, n_words)(W, wtail, cs)
    esamp = eflat.reshape(B, F)
    wsamp = wflat.reshape(B, F)
    return _make_join_kernel(B, F)(inv_e, inv_w, esamp, wsamp)


# speedup vs baseline: 1.0801x; 1.0035x over previous
import functools

import jax
import jax.numpy as jnp
from jax import lax
from jax.experimental import pallas as pl
from jax.experimental.pallas import tpu as pltpu
from jax.experimental.pallas import tpu_sc as plsc

NC = 2
NS = 16
NW = NC * NS
LANES = 16
CHUNK = 128
WT = 12
WCOLS = WT * 128
TAILPAD = 128


def _make_extract_kernel(B, F, n_cols):
    per_w = B // NW
    n_blk = per_w // LANES
    mesh = plsc.VectorSubcoreMesh(core_axis_name="c", subcore_axis_name="s")

    n_tiles = n_cols // CHUNK
    main_c = n_tiles * CHUNK
    t_max = max(n_tiles - WT, 0)

    @functools.partial(
        pl.kernel,
        out_type=jax.ShapeDtypeStruct((B * F,), jnp.float32),
        mesh=mesh,
        scratch_types=[
            pltpu.VMEM((F, WCOLS), jnp.float32),
            pltpu.VMEM((per_w * F // 2,), jnp.float32),
            pltpu.VMEM((per_w,), jnp.int32),
            pltpu.VMEM((F, TAILPAD), jnp.float32),
        ],
        compiler_params=pltpu.CompilerParams(needs_layout_passes=False,
                                             use_tc_tiling_on_sc=True),
    )
    def extract(tab, tail, srt, out, wbuf, stg, sc_v, tbuf):
        cid = lax.axis_index("c")
        sid = lax.axis_index("s")
        wid = sid * NC + cid
        iota = lax.broadcasted_iota(jnp.int32, (LANES,), 0)

        pltpu.sync_copy(srt.at[pl.ds(wid * per_w, per_w)], sc_v)
        pltpu.sync_copy(tail, tbuf)
        half_blk = n_blk // 2

        for h in range(2):
            base_blk = h * half_blk

            def blk_body(blk, t0_cur, base_blk=base_blk):
                cols16 = sc_v[pl.ds(blk * LANES, LANES)]
                sidx0 = ((blk - base_blk) * LANES + iota) * F
                tmask = cols16 >= main_c
                n_tail = jnp.sum(tmask.astype(jnp.int32))

                @pl.when(n_tail > 0)
                def _():
                    tloc = jnp.clip(cols16 - main_c, 0, TAILPAD - 1)
                    fv = jnp.zeros((LANES,), jnp.int32)
                    sidx = sidx0
                    for _f in range(F):
                        tv = plsc.load_gather(tbuf, [fv, tloc], mask=tmask)
                        plsc.store_scatter(stg, [sidx], tv, mask=tmask)
                        fv = fv + 1
                        sidx = sidx + 1

                def wcond(state):
                    done, _ = state
                    return jnp.sum(done.astype(jnp.int32)) < LANES

                def wbody(state):
                    done, t0_old = state
                    cbig = jnp.where(done, jnp.int32(1 << 30), cols16)
                    cmin = jnp.min(cbig)
                    hit = jnp.logical_and(cmin >= t0_old * CHUNK,
                                          cmin < t0_old * CHUNK + WCOLS)
                    t0 = jnp.where(hit, t0_old,
                                   jnp.minimum(cmin >> 7, t_max))
                    c0 = t0 * CHUNK

                    @pl.when(jnp.logical_not(hit))
                    def _():
                        pltpu.sync_copy(tab.at[:, pl.ds(c0, WCOLS)], wbuf)

                    cover = jnp.logical_and(
                        jnp.logical_and(cols16 >= c0, cols16 < c0 + WCOLS),
                        jnp.logical_not(tmask))
                    lc = jnp.clip(cols16 - c0, 0, WCOLS - 1)
                    fv = jnp.zeros((LANES,), jnp.int32)
                    sidx = sidx0
                    for _f in range(F):
                        v = plsc.load_gather(wbuf, [fv, lc], mask=cover)
                        plsc.store_scatter(stg, [sidx], v, mask=cover)
                        fv = fv + 1
                        sidx = sidx + 1
                    return jnp.logical_or(done, cover), t0

                _, t0_out = lax.while_loop(wcond, wbody, (tmask, t0_cur))
                return t0_out

            t0_fin = lax.fori_loop(base_blk, base_blk + half_blk, blk_body,
                                   jnp.int32(-(1 << 20)) if h == 0 else t0_c)
            t0_c = t0_fin
            pltpu.sync_copy(
                stg,
                out.at[pl.ds((wid * per_w + h * per_w // 2) * F,
                             per_w * F // 2)])

    return extract


def _make_join_kernel(B, F):
    per_w = B // NW
    n_chunks = per_w // CHUNK
    mesh = plsc.VectorSubcoreMesh(core_axis_name="c", subcore_axis_name="s")

    @functools.partial(
        pl.kernel,
        out_type=jax.ShapeDtypeStruct((B,), jnp.float32),
        mesh=mesh,
        scratch_types=[
            pltpu.VMEM((per_w,), jnp.int32),
            pltpu.VMEM((per_w,), jnp.int32),
            pltpu.VMEM((per_w, F), jnp.float32),
            pltpu.VMEM((per_w, F), jnp.float32),
            pltpu.VMEM((per_w,), jnp.float32),
            pltpu.SemaphoreType.DMA,
        ],
        compiler_params=pltpu.CompilerParams(needs_layout_passes=False,
                                             use_tc_tiling_on_sc=False),
    )
    def join(eidx_hbm, widx_hbm, es_hbm, ws_hbm, out_hbm,
             eidx_v, widx_v, e_v, w_v, out_v, sem):
        cid = lax.axis_index("c")
        sid = lax.axis_index("s")
        wid = sid * NC + cid
        iota = lax.broadcasted_iota(jnp.int32, (LANES,), 0)
        base = wid * per_w

        pltpu.sync_copy(eidx_hbm.at[pl.ds(base, per_w)], eidx_v)
        pltpu.sync_copy(widx_hbm.at[pl.ds(base, per_w)], widx_v)
        cps = []
        for g in range(n_chunks):
            o = g * CHUNK
            cps.append(pltpu.async_copy(
                es_hbm.at[eidx_v.at[pl.ds(o, CHUNK)]],
                e_v.at[pl.ds(o, CHUNK), :], sem))
            cps.append(pltpu.async_copy(
                ws_hbm.at[widx_v.at[pl.ds(o, CHUNK)]],
                w_v.at[pl.ds(o, CHUNK), :], sem))
        for cp in cps:
            cp.wait()

        def block_body(bi, _):
            b0 = bi * LANES
            rows16 = b0 + iota
            acc = jnp.zeros((LANES,), jnp.float32)
            cols16 = jnp.zeros((LANES,), jnp.int32)
            for _f in range(F):
                e_vals = plsc.load_gather(e_v, [rows16, cols16])
                w_vals = plsc.load_gather(w_v, [rows16, cols16])
                acc = acc + e_vals * w_vals
                cols16 = cols16 + 1
            out_v[pl.ds(b0, LANES)] = acc
            return 0

        lax.fori_loop(0, per_w // LANES, block_body, 0)
        pltpu.sync_copy(out_v, out_hbm.at[pl.ds(base, per_w)])

    return join


def _padded_tail(tab_fmajor, main_c):
    tail = tab_fmajor[:, main_c:]
    return jnp.pad(tail, ((0, 0), (0, TAILPAD - tail.shape[1])))


def kernel(batch, E, W):
    B = batch.shape[0]
    n_ent, F = E.shape
    n_words = W.shape[1]
    rows = batch[:, 0].astype(jnp.int32)
    cols = batch[:, 1].astype(jnp.int32)

    ar = jnp.arange(B, dtype=jnp.int32)
    rs, order_e = lax.sort((rows, ar), num_keys=1)
    cs, order_w = lax.sort((cols, ar), num_keys=1)
    inv_e = jnp.zeros((B,), jnp.int32).at[order_e].set(ar)
    inv_w = jnp.zeros((B,), jnp.int32).at[order_w].set(ar)

    ET = E.T
    main_e = (n_ent // CHUNK) * CHUNK
    main_w = (n_words // CHUNK) * CHUNK
    etail = _padded_tail(ET, main_e)
    wtail = _padded_tail(W, main_w)

    eflat = _make_extract_kernel(B, F, n_ent)(ET, etail, rs)
    wflat = _make_extract_kernel(B, F, n_words)(W, wtail, cs)
    esamp = eflat.reshape(B, F)
    wsamp = wflat.reshape(B, F)
    return _make_join_kernel(B, F)(inv_e, inv_w, esamp, wsamp)


# speedup vs baseline: 1.0809x; 1.0007x over previous
import functools

import jax
import jax.numpy as jnp
from jax import lax
from jax.experimental import pallas as pl
from jax.experimental.pallas import tpu as pltpu
from jax.experimental.pallas import tpu_sc as plsc

NC = 2
NS = 16
NW = NC * NS
LANES = 16
CHUNK = 128
WT = 12
WCOLS = WT * 128
TAILPAD = 128


def _make_extract_kernel(B, F, n_cols):
    per_w = B // NW
    n_blk = per_w // LANES
    mesh = plsc.VectorSubcoreMesh(core_axis_name="c", subcore_axis_name="s")

    n_tiles = n_cols // CHUNK
    main_c = n_tiles * CHUNK
    t_max = max(n_tiles - WT, 0)

    @functools.partial(
        pl.kernel,
        out_type=jax.ShapeDtypeStruct((B * F,), jnp.float32),
        mesh=mesh,
        scratch_types=[
            pltpu.VMEM((F, WCOLS), jnp.float32),
            pltpu.VMEM((per_w * F // 2,), jnp.float32),
            pltpu.VMEM((per_w,), jnp.int32),
            pltpu.VMEM((F, TAILPAD), jnp.float32),
        ],
        compiler_params=pltpu.CompilerParams(needs_layout_passes=False,
                                             use_tc_tiling_on_sc=True),
    )
    def extract(tab, tail, srt, out, wbuf, stg, sc_v, tbuf):
        cid = lax.axis_index("c")
        sid = lax.axis_index("s")
        wid = sid * NC + cid
        iota = lax.broadcasted_iota(jnp.int32, (LANES,), 0)

        pltpu.sync_copy(srt.at[pl.ds(wid * per_w, per_w)], sc_v)
        pltpu.sync_copy(tail, tbuf)
        half_blk = n_blk // 2

        for h in range(2):
            base_blk = h * half_blk

            def blk_body(blk, t0_cur, base_blk=base_blk):
                cols16 = sc_v[pl.ds(blk * LANES, LANES)]
                sidx0 = ((blk - base_blk) * LANES + iota) * F
                tmask = cols16 >= main_c
                n_tail = jnp.sum(tmask.astype(jnp.int32))

                @pl.when(n_tail > 0)
                def _():
                    tloc = jnp.clip(cols16 - main_c, 0, TAILPAD - 1)
                    fv = jnp.zeros((LANES,), jnp.int32)
                    sidx = sidx0
                    for _f in range(F):
                        tv = plsc.load_gather(tbuf, [fv, tloc], mask=tmask)
                        plsc.store_scatter(stg, [sidx], tv, mask=tmask)
                        fv = fv + 1
                        sidx = sidx + 1

                def wcond(state):
                    done, _ = state
                    return jnp.sum(done.astype(jnp.int32)) < LANES

                def wbody(state):
                    done, t0_old = state
                    cbig = jnp.where(done, jnp.int32(1 << 30), cols16)
                    cmin = jnp.min(cbig)
                    hit = jnp.logical_and(cmin >= t0_old * CHUNK,
                                          cmin < t0_old * CHUNK + WCOLS)
                    t0 = jnp.where(hit, t0_old,
                                   jnp.minimum(cmin >> 7, t_max))
                    c0 = t0 * CHUNK

                    @pl.when(jnp.logical_not(hit))
                    def _():
                        pltpu.sync_copy(tab.at[:, pl.ds(c0, WCOLS)], wbuf)

                    cover = jnp.logical_and(
                        jnp.logical_and(cols16 >= c0, cols16 < c0 + WCOLS),
                        jnp.logical_not(tmask))
                    lc = jnp.clip(cols16 - c0, 0, WCOLS - 1)
                    fv = jnp.zeros((LANES,), jnp.int32)
                    sidx = sidx0
                    for _f in range(F):
                        v = plsc.load_gather(wbuf, [fv, lc], mask=cover)
                        plsc.store_scatter(stg, [sidx], v, mask=cover)
                        fv = fv + 1
                        sidx = sidx + 1
                    return jnp.logical_or(done, cover), t0

                _, t0_out = lax.while_loop(wcond, wbody, (tmask, t0_cur))
                return t0_out

            t0_fin = lax.fori_loop(base_blk, base_blk + half_blk, blk_body,
                                   jnp.int32(-(1 << 20)) if h == 0 else t0_c)
            t0_c = t0_fin
            pltpu.sync_copy(
                stg,
                out.at[pl.ds((wid * per_w + h * per_w // 2) * F,
                             per_w * F // 2)])

    return extract


def _make_join_kernel(B, F):
    per_w = B // NW
    n_chunks = per_w // CHUNK
    mesh = plsc.VectorSubcoreMesh(core_axis_name="c", subcore_axis_name="s")

    @functools.partial(
        pl.kernel,
        out_type=jax.ShapeDtypeStruct((B,), jnp.float32),
        mesh=mesh,
        scratch_types=[
            pltpu.VMEM((per_w,), jnp.int32),
            pltpu.VMEM((per_w,), jnp.int32),
            pltpu.VMEM((per_w, F), jnp.float32),
            pltpu.VMEM((per_w, F), jnp.float32),
            pltpu.VMEM((per_w,), jnp.float32),
            pltpu.SemaphoreType.DMA,
        ],
        compiler_params=pltpu.CompilerParams(needs_layout_passes=False,
                                             use_tc_tiling_on_sc=False),
    )
    def join(eidx_hbm, widx_hbm, es_hbm, ws_hbm, out_hbm,
             eidx_v, widx_v, e_v, w_v, out_v, sem):
        cid = lax.axis_index("c")
        sid = lax.axis_index("s")
        wid = sid * NC + cid
        iota = lax.broadcasted_iota(jnp.int32, (LANES,), 0)
        base = wid * per_w

        cpi_e = pltpu.async_copy(eidx_hbm.at[pl.ds(base, per_w)], eidx_v, sem)
        cpi_w = pltpu.async_copy(widx_hbm.at[pl.ds(base, per_w)], widx_v, sem)
        cpi_e.wait()
        cpi_w.wait()
        cps = []
        for g in range(n_chunks):
            o = g * CHUNK
            cps.append(pltpu.async_copy(
                es_hbm.at[eidx_v.at[pl.ds(o, CHUNK)]],
                e_v.at[pl.ds(o, CHUNK), :], sem))
            cps.append(pltpu.async_copy(
                ws_hbm.at[widx_v.at[pl.ds(o, CHUNK)]],
                w_v.at[pl.ds(o, CHUNK), :], sem))
        for cp in cps:
            cp.wait()

        def block_body(bi, _):
            b0 = bi * LANES
            rows16 = b0 + iota
            acc = jnp.zeros((LANES,), jnp.float32)
            cols16 = jnp.zeros((LANES,), jnp.int32)
            for _f in range(F):
                e_vals = plsc.load_gather(e_v, [rows16, cols16])
                w_vals = plsc.load_gather(w_v, [rows16, cols16])
                acc = acc + e_vals * w_vals
                cols16 = cols16 + 1
            out_v[pl.ds(b0, LANES)] = acc
            return 0

        lax.fori_loop(0, per_w // LANES, block_body, 0)
        pltpu.sync_copy(out_v, out_hbm.at[pl.ds(base, per_w)])

    return join


def _padded_tail(tab_fmajor, main_c):
    tail = tab_fmajor[:, main_c:]
    return jnp.pad(tail, ((0, 0), (0, TAILPAD - tail.shape[1])))


def kernel(batch, E, W):
    B = batch.shape[0]
    n_ent, F = E.shape
    n_words = W.shape[1]
    rows = batch[:, 0].astype(jnp.int32)
    cols = batch[:, 1].astype(jnp.int32)

    ar = jnp.arange(B, dtype=jnp.int32)
    rs, order_e = lax.sort((rows, ar), num_keys=1)
    cs, order_w = lax.sort((cols, ar), num_keys=1)
    inv_e = jnp.zeros((B,), jnp.int32).at[order_e].set(ar)
    inv_w = jnp.zeros((B,), jnp.int32).at[order_w].set(ar)

    ET = E.T
    main_e = (n_ent // CHUNK) * CHUNK
    main_w = (n_words // CHUNK) * CHUNK
    etail = _padded_tail(ET, main_e)
    wtail = _padded_tail(W, main_w)

    eflat = _make_extract_kernel(B, F, n_ent)(ET, etail, rs)
    wflat = _make_extract_kernel(B, F, n_words)(W, wtail, cs)
    esamp = eflat.reshape(B, F)
    wsamp = wflat.reshape(B, F)
    return _make_join_kernel(B, F)(inv_e, inv_w, esamp, wsamp)
